# baseline (speedup 1.0000x reference)
import functools

import jax
import jax.numpy as jnp
import numpy as np
from jax import lax
from jax.experimental import pallas as pl
from jax.experimental.pallas import tpu as pltpu
from jax.experimental.pallas import tpu_sc as plsc

B, L, V, D = 1024, 200, 100000, 128
N = B * L
NC, NS = 2, 16
NW = NC * NS
PER_W = N // NW
C = 128
NCHUNK = PER_W // C
LANES = 16


def _pos_embed_np():
    pos = np.arange(L, dtype=np.float32)[:, None]
    ids = np.arange(D)
    even = (ids % 2) == 0
    exponent = np.where(even, ids, ids - 1).astype(np.float32) / D
    angle = pos / np.power(10000.0, exponent)[None, :]
    pe = np.where(even[None, :], np.sin(angle), np.cos(angle))
    return pe.astype(np.float32)


_MESH = plsc.VectorSubcoreMesh(
    core_axis_name="c", subcore_axis_name="s", num_cores=NC, num_subcores=NS
)


@functools.partial(
    pl.kernel,
    out_type=jax.ShapeDtypeStruct((N, D), jnp.float32),
    mesh=_MESH,
    scratch_types=[
        pltpu.VMEM((C,), jnp.int32),
        pltpu.VMEM((C,), jnp.int32),
        pltpu.VMEM((C, D), jnp.float32),
        pltpu.VMEM((L, D), jnp.float32),
        pltpu.VMEM((3, D), jnp.float32),
        pltpu.SemaphoreType.DMA,
    ],
)
def _embed_kernel(tok_tab, pos_tab, seg_tab, tok_idx, seg_lab, out,
                  ti_v, sg_v, bt_v, pos_res, seg_res, sem_t):
    wid = lax.axis_index("s") * NC + lax.axis_index("c")
    wbase = wid * PER_W
    pltpu.sync_copy(pos_tab, pos_res)
    pltpu.sync_copy(seg_tab, seg_res)

    @pl.loop(0, NCHUNK)
    def _chunk(c):
        base = wbase + c * C
        pltpu.sync_copy(tok_idx.at[pl.ds(base, C)], ti_v)
        pltpu.sync_copy(seg_lab.at[pl.ds(base, C)], sg_v)
        pltpu.async_copy(tok_tab.at[ti_v], bt_v, sem_t).wait()

        @pl.loop(0, C // LANES)
        def _add_grp(g):
            sgv = sg_v[pl.ds(g * LANES, LANES)]
            for k in range(LANES):
                r = g * LANES + k
                sg = sgv[k]
                lr = lax.rem(base + r, L)
                for j in range(D // LANES):
                    sl = pl.ds(j * LANES, LANES)
                    bt_v[r, sl] += pos_res[lr, sl] + seg_res[sg, sl]

        pltpu.sync_copy(bt_v, out.at[pl.ds(base, C)])


def kernel(input_tensor, segment_label, token_table, segm_table):
    pe = jnp.asarray(_pos_embed_np())
    tok_idx = input_tensor.reshape(-1).astype(jnp.int32)
    seg_lab = segment_label.reshape(-1).astype(jnp.int32)
    out = _embed_kernel(token_table, pe, segm_table, tok_idx, seg_lab)
    return out.reshape(B, L, D)


# speedup vs baseline: 2.6825x; 2.6825x over previous
import functools

import jax
import jax.numpy as jnp
import numpy as np
from jax import lax
from jax.experimental import pallas as pl
from jax.experimental.pallas import tpu as pltpu
from jax.experimental.pallas import tpu_sc as plsc

B, L, V, D = 1024, 200, 100000, 128
N = B * L
NC, NS = 2, 16
NW = NC * NS
PER_W = N // NW
C = 128
NCHUNK = PER_W // C
LANES = 16


def _pos_embed_np():
    pos = np.arange(L, dtype=np.float32)[:, None]
    ids = np.arange(D)
    even = (ids % 2) == 0
    exponent = np.where(even, ids, ids - 1).astype(np.float32) / D
    angle = pos / np.power(10000.0, exponent)[None, :]
    pe = np.where(even[None, :], np.sin(angle), np.cos(angle))
    return pe.astype(np.float32)


_MESH = plsc.VectorSubcoreMesh(
    core_axis_name="c", subcore_axis_name="s", num_cores=NC, num_subcores=NS
)


@functools.partial(
    pl.kernel,
    out_type=jax.ShapeDtypeStruct((N, D), jnp.float32),
    mesh=_MESH,
    scratch_types=[
        pltpu.VMEM((PER_W,), jnp.int32),
        pltpu.VMEM((PER_W,), jnp.int32),
        pltpu.VMEM((2, C, D), jnp.float32),
        pltpu.VMEM((2, C, D), jnp.float32),
        pltpu.SemaphoreType.DMA,
        pltpu.SemaphoreType.DMA,
        pltpu.SemaphoreType.DMA,
        pltpu.SemaphoreType.DMA,
        pltpu.SemaphoreType.DMA,
        pltpu.SemaphoreType.DMA,
    ],
)
def _embed_kernel(tok_tab, combo_tab, tok_idx, seg_lab, out,
                  ti_v, ci_v, bt, bc,
                  sem_t0, sem_t1, sem_c0, sem_c1, sem_o0, sem_o1):
    sem_t = (sem_t0, sem_t1)
    sem_c = (sem_c0, sem_c1)
    sem_o = (sem_o0, sem_o1)
    wid = lax.axis_index("s") * NC + lax.axis_index("c")
    wbase = wid * PER_W

    pltpu.sync_copy(tok_idx.at[pl.ds(wbase, PER_W)], ti_v)
    pltpu.sync_copy(seg_lab.at[pl.ds(wbase, PER_W)], ci_v)

    @pl.loop(0, PER_W // LANES)
    def _mkidx(j):
        sl = pl.ds(j * LANES, LANES)
        rows = wbase + j * LANES + lax.iota(jnp.int32, 16)
        ci_v[sl] = ci_v[sl] * L + lax.rem(rows, L)

    def issue(cn, s):
        pltpu.async_copy(tok_tab.at[ti_v.at[pl.ds(cn * C, C)]], bt.at[s], sem_t[s])
        pltpu.async_copy(combo_tab.at[ci_v.at[pl.ds(cn * C, C)]], bc.at[s], sem_c[s])

    def wait_gathers(cn, s):
        pltpu.make_async_copy(tok_tab.at[ti_v.at[pl.ds(cn * C, C)]], bt.at[s], sem_t[s]).wait()
        pltpu.make_async_copy(combo_tab.at[ci_v.at[pl.ds(cn * C, C)]], bc.at[s], sem_c[s]).wait()

    def wait_scatter(cn, s):
        pltpu.make_async_copy(
            bt.at[s], out.at[pl.ds(wbase + cn * C, C)], sem_o[s]).wait()

    issue(0, 0)

    @pl.loop(0, NCHUNK // 2)
    def _pair(g):
        for b in (0, 1):
            cn = 2 * g + b
            nb = 1 - b

            @pl.when(cn + 1 < NCHUNK)
            def _issue_next():
                @pl.when(cn >= 1)
                def _drain_prev():
                    wait_scatter(cn - 1, nb)
                issue(cn + 1, nb)

            wait_gathers(cn, b)
            bts = bt.at[b]
            bcs = bc.at[b]

            @pl.loop(0, C)
            def _add_row(r):
                for j in range(D // LANES):
                    sl = pl.ds(j * LANES, LANES)
                    bts[r, sl] += bcs[r, sl]

            pltpu.async_copy(
                bts, out.at[pl.ds(wbase + cn * C, C)], sem_o[b])

    wait_scatter(NCHUNK - 2, 0)
    wait_scatter(NCHUNK - 1, 1)


def kernel(input_tensor, segment_label, token_table, segm_table):
    pe = jnp.asarray(_pos_embed_np())
    combo_tab = (segm_table[:, None, :] + pe[None, :, :]).reshape(3 * L, D)
    tok_idx = input_tensor.reshape(-1).astype(jnp.int32)
    seg_lab = segment_label.reshape(-1).astype(jnp.int32)
    out = _embed_kernel(token_table, combo_tab, tok_idx, seg_lab)
    return out.reshape(B, L, D)


# speedup vs baseline: 4.0666x; 1.5160x over previous
import functools

import jax
import jax.numpy as jnp
import numpy as np
from jax import lax
from jax.experimental import pallas as pl
from jax.experimental.pallas import tpu as pltpu
from jax.experimental.pallas import tpu_sc as plsc

B, L, V, D = 1024, 200, 100000, 128
N = B * L
NC, NS = 2, 16
NW = NC * NS
PER_W = N // NW
C = 128
NCHUNK = PER_W // C
LANES = 16
CROWS = 3 * L
CPAD = 640
CSTAGE = CPAD // NS


def _pos_embed_np():
    pos = np.arange(L, dtype=np.float32)[:, None]
    ids = np.arange(D)
    even = (ids % 2) == 0
    exponent = np.where(even, ids, ids - 1).astype(np.float32) / D
    angle = pos / np.power(10000.0, exponent)[None, :]
    pe = np.where(even[None, :], np.sin(angle), np.cos(angle))
    return pe.astype(np.float32)


_MESH = plsc.VectorSubcoreMesh(
    core_axis_name="c", subcore_axis_name="s", num_cores=NC, num_subcores=NS
)


@functools.partial(
    pl.kernel,
    out_type=jax.ShapeDtypeStruct((N, D), jnp.float32),
    mesh=_MESH,
    scratch_types=[
        pltpu.VMEM((PER_W,), jnp.int32),
        pltpu.VMEM((PER_W,), jnp.int32),
        pltpu.VMEM((2, C, D), jnp.float32),
        pltpu.VMEM((2, C, D), jnp.float32),
        pltpu.VMEM_SHARED((CPAD, D), jnp.float32),
        pltpu.SemaphoreType.DMA,
        pltpu.SemaphoreType.DMA,
        pltpu.SemaphoreType.DMA,
        pltpu.SemaphoreType.DMA,
        pltpu.SemaphoreType.DMA,
        pltpu.SemaphoreType.DMA,
    ],
)
def _embed_kernel(tok_tab, combo_tab, tok_idx, seg_lab, out,
                  ti_v, ci_v, bt, bc, combo_sh,
                  sem_t0, sem_t1, sem_c0, sem_c1, sem_o0, sem_o1):
    sem_t = (sem_t0, sem_t1)
    sem_c = (sem_c0, sem_c1)
    sem_o = (sem_o0, sem_o1)
    sid = lax.axis_index("s")
    wid = sid * NC + lax.axis_index("c")
    wbase = wid * PER_W

    pltpu.sync_copy(combo_tab.at[pl.ds(sid * CSTAGE, CSTAGE)],
                    bc.at[0, pl.ds(0, CSTAGE)])
    pltpu.sync_copy(bc.at[0, pl.ds(0, CSTAGE)],
                    combo_sh.at[pl.ds(sid * CSTAGE, CSTAGE)])

    pltpu.sync_copy(tok_idx.at[pl.ds(wbase, PER_W)], ti_v)
    pltpu.sync_copy(seg_lab.at[pl.ds(wbase, PER_W)], ci_v)

    @pl.loop(0, PER_W // LANES)
    def _mkidx(j):
        sl = pl.ds(j * LANES, LANES)
        rows = wbase + j * LANES + lax.iota(jnp.int32, 16)
        ci_v[sl] = ci_v[sl] * L + lax.rem(rows, L)

    def issue(cn, s):
        pltpu.async_copy(tok_tab.at[ti_v.at[pl.ds(cn * C, C)]], bt.at[s], sem_t[s])
        pltpu.async_copy(combo_sh.at[ci_v.at[pl.ds(cn * C, C)]], bc.at[s], sem_c[s])

    def wait_gathers(cn, s):
        pltpu.make_async_copy(tok_tab.at[ti_v.at[pl.ds(cn * C, C)]], bt.at[s], sem_t[s]).wait()
        pltpu.make_async_copy(combo_sh.at[ci_v.at[pl.ds(cn * C, C)]], bc.at[s], sem_c[s]).wait()

    def wait_scatter(cn, s):
        pltpu.make_async_copy(
            bt.at[s], out.at[pl.ds(wbase + cn * C, C)], sem_o[s]).wait()

    plsc.subcore_barrier()

    issue(0, 0)

    @pl.loop(0, NCHUNK // 2)
    def _pair(g):
        for b in (0, 1):
            cn = 2 * g + b
            nb = 1 - b

            @pl.when(cn + 1 < NCHUNK)
            def _issue_next():
                @pl.when(cn >= 1)
                def _drain_prev():
                    wait_scatter(cn - 1, nb)
                issue(cn + 1, nb)

            wait_gathers(cn, b)
            bts = bt.at[b]
            bcs = bc.at[b]

            @pl.loop(0, C)
            def _add_row(r):
                for j in range(D // LANES):
                    sl = pl.ds(j * LANES, LANES)
                    bts[r, sl] += bcs[r, sl]

            pltpu.async_copy(
                bts, out.at[pl.ds(wbase + cn * C, C)], sem_o[b])

    wait_scatter(NCHUNK - 2, 0)
    wait_scatter(NCHUNK - 1, 1)


def kernel(input_tensor, segment_label, token_table, segm_table):
    pe = jnp.asarray(_pos_embed_np())
    combo_tab = (segm_table[:, None, :] + pe[None, :, :]).reshape(CROWS, D)
    combo_tab = jnp.pad(combo_tab, ((0, CPAD - CROWS), (0, 0)))
    tok_idx = input_tensor.reshape(-1).astype(jnp.int32)
    seg_lab = segment_label.reshape(-1).astype(jnp.int32)
    out = _embed_kernel(token_table, combo_tab, tok_idx, seg_lab)
    return out.reshape(B, L, D)


# speedup vs baseline: 4.5224x; 1.1121x over previous
import functools

import jax
import jax.numpy as jnp
import numpy as np
from jax import lax
from jax.experimental import pallas as pl
from jax.experimental.pallas import tpu as pltpu
from jax.experimental.pallas import tpu_sc as plsc

B, L, V, D = 1024, 200, 100000, 128
N = B * L
NC, NS = 2, 16
NW = NC * NS
PER_W = N // NW
C = 128
NCHUNK = PER_W // C
LANES = 16
CROWS = 3 * L
CPAD = 640
CSTAGE = CPAD // NS


def _pos_embed_np():
    pos = np.arange(L, dtype=np.float32)[:, None]
    ids = np.arange(D)
    even = (ids % 2) == 0
    exponent = np.where(even, ids, ids - 1).astype(np.float32) / D
    angle = pos / np.power(10000.0, exponent)[None, :]
    pe = np.where(even[None, :], np.sin(angle), np.cos(angle))
    return pe.astype(np.float32)


_MESH = plsc.VectorSubcoreMesh(
    core_axis_name="c", subcore_axis_name="s", num_cores=NC, num_subcores=NS
)


@functools.partial(
    pl.kernel,
    out_type=jax.ShapeDtypeStruct((N, D), jnp.float32),
    mesh=_MESH,
    scratch_types=[
        pltpu.VMEM((PER_W,), jnp.int32),
        pltpu.VMEM((PER_W,), jnp.int32),
        pltpu.VMEM((2, C, D), jnp.float32),
        pltpu.VMEM((2, C, D), jnp.float32),
        pltpu.VMEM((2, C, D), jnp.float32),
        pltpu.VMEM_SHARED((CPAD, D), jnp.float32),
        pltpu.SemaphoreType.DMA,
        pltpu.SemaphoreType.DMA,
        pltpu.SemaphoreType.DMA,
        pltpu.SemaphoreType.DMA,
        pltpu.SemaphoreType.DMA,
        pltpu.SemaphoreType.DMA,
    ],
)
def _embed_kernel(tok_tab, combo_tab, tok_idx, seg_lab, out,
                  ti_v, ci_v, bt, bc, bo, combo_sh,
                  sem_t0, sem_t1, sem_c0, sem_c1, sem_o0, sem_o1):
    sem_t = (sem_t0, sem_t1)
    sem_c = (sem_c0, sem_c1)
    sem_o = (sem_o0, sem_o1)
    sid = lax.axis_index("s")
    wid = sid * NC + lax.axis_index("c")
    wbase = wid * PER_W

    pltpu.sync_copy(combo_tab.at[pl.ds(sid * CSTAGE, CSTAGE)],
                    bc.at[0, pl.ds(0, CSTAGE)])
    pltpu.sync_copy(bc.at[0, pl.ds(0, CSTAGE)],
                    combo_sh.at[pl.ds(sid * CSTAGE, CSTAGE)])

    pltpu.sync_copy(tok_idx.at[pl.ds(wbase, PER_W)], ti_v)
    pltpu.sync_copy(seg_lab.at[pl.ds(wbase, PER_W)], ci_v)

    @pl.loop(0, PER_W // LANES)
    def _mkidx(j):
        sl = pl.ds(j * LANES, LANES)
        rows = wbase + j * LANES + lax.iota(jnp.int32, 16)
        ci_v[sl] = ci_v[sl] * L + lax.rem(rows, L)

    def issue(cn, s):
        pltpu.async_copy(tok_tab.at[ti_v.at[pl.ds(cn * C, C)]], bt.at[s], sem_t[s])
        pltpu.async_copy(combo_sh.at[ci_v.at[pl.ds(cn * C, C)]], bc.at[s], sem_c[s])

    def wait_gathers(cn, s):
        pltpu.make_async_copy(tok_tab.at[ti_v.at[pl.ds(cn * C, C)]], bt.at[s], sem_t[s]).wait()
        pltpu.make_async_copy(combo_sh.at[ci_v.at[pl.ds(cn * C, C)]], bc.at[s], sem_c[s]).wait()

    def wait_scatter(cn, s):
        pltpu.make_async_copy(
            bo.at[s], out.at[pl.ds(wbase + cn * C, C)], sem_o[s]).wait()

    plsc.subcore_barrier()

    issue(0, 0)

    @pl.loop(0, NCHUNK // 2)
    def _pair(g):
        for b in (0, 1):
            cn = 2 * g + b
            nb = 1 - b

            @pl.when(cn + 1 < NCHUNK)
            def _issue_next():
                issue(cn + 1, nb)

            wait_gathers(cn, b)

            @pl.when(cn >= 2)
            def _drain_bo():
                wait_scatter(cn - 2, b)

            bts = bt.at[b]
            bcs = bc.at[b]
            bos = bo.at[b]

            @pl.loop(0, C)
            def _add_row(r):
                for j in range(D // LANES):
                    sl = pl.ds(j * LANES, LANES)
                    bos[r, sl] = bts[r, sl] + bcs[r, sl]

            pltpu.async_copy(
                bos, out.at[pl.ds(wbase + cn * C, C)], sem_o[b])

    wait_scatter(NCHUNK - 2, 0)
    wait_scatter(NCHUNK - 1, 1)


def kernel(input_tensor, segment_label, token_table, segm_table):
    pe = jnp.asarray(_pos_embed_np())
    combo_tab = (segm_table[:, None, :] + pe[None, :, :]).reshape(CROWS, D)
    combo_tab = jnp.pad(combo_tab, ((0, CPAD - CROWS), (0, 0)))
    tok_idx = input_tensor.reshape(-1).astype(jnp.int32)
    seg_lab = segment_label.reshape(-1).astype(jnp.int32)
    out = _embed_kernel(token_table, combo_tab, tok_idx, seg_lab)
    return out.reshape(B, L, D)
